# baseline (speedup 1.0000x reference)
import functools

import jax
import jax.numpy as jnp
from jax import lax
from jax.experimental import pallas as pl
from jax.experimental.pallas import tpu as pltpu
from jax.experimental.pallas import tpu_sc as plsc

N = 50000
E = 800000
H = 64
E_PAD = 819200
NC, NS = 2, 16
NW = NC * NS
HALF = N // 2
TBL = 25024
TRASH = TBL - 8
CH = 512
CHS = 128


def _gather_rows(table, idx2, n_rows):
    epw = n_rows // NW
    nch = epw // CH
    nchh = nch // 2
    jj = CH // 128
    mesh = plsc.VectorSubcoreMesh(core_axis_name="c", subcore_axis_name="s")

    @functools.partial(
        pl.kernel,
        mesh=mesh,
        out_type=jax.ShapeDtypeStruct((n_rows, H), jnp.float32),
        scratch_types=[
            pltpu.VMEM((2, jj, 128), jnp.int32),
            pltpu.VMEM((2, CH, H), jnp.float32),
            pltpu.SemaphoreType.DMA,
        ],
        compiler_params=pltpu.CompilerParams(use_tc_tiling_on_sc=False),
    )
    def k(table_hbm, idx_hbm, out_hbm, idx_v, rows_v, semg):
        c = lax.axis_index("c")
        s = lax.axis_index("s")
        wid = s * NC + c
        base = wid * epw
        irow = wid * (epw // 128)

        def load_idx(i, b):
            pltpu.sync_copy(idx_hbm.at[pl.ds(irow + i * jj, jj)], idx_v.at[b])

        def fire(b):
            for j in range(jj):
                pltpu.async_copy(table_hbm.at[idx_v.at[b, j]],
                                 rows_v.at[b, pl.ds(j * 128, 128)], semg)

        def drain(b):
            for j in range(jj):
                pltpu.make_async_copy(table_hbm.at[idx_v.at[b, j]],
                                      rows_v.at[b, pl.ds(j * 128, 128)],
                                      semg).wait()

        def store(i, b):
            pltpu.sync_copy(rows_v.at[b], out_hbm.at[pl.ds(base + i * CH, CH)])

        load_idx(0, 0)
        fire(0)

        def body2(kk, carry):
            i0 = 2 * kk
            load_idx(i0 + 1, 1)
            drain(0)
            fire(1)
            store(i0, 0)

            @pl.when(kk < nchh - 1)
            def _():
                load_idx(i0 + 2, 0)
                fire(0)

            drain(1)
            store(i0 + 1, 1)
            return carry

        lax.fori_loop(0, nchh, body2, 0)

    return k(table, idx2)


def _scatter_add(msg, dst2, zeros_tbl):
    ept = E_PAD // NS
    nch = ept // CHS
    nchh = nch // 2
    rpt = TBL // NS
    mesh = plsc.VectorSubcoreMesh(core_axis_name="c", subcore_axis_name="s")

    @functools.partial(
        pl.kernel,
        mesh=mesh,
        out_type=jax.ShapeDtypeStruct((NC, TBL, H), jnp.float32),
        scratch_types=[
            pltpu.VMEM((2, 1, 128), jnp.int32),
            pltpu.VMEM((2, CHS, H), jnp.float32),
            pltpu.VMEM_SHARED((TBL, H), jnp.float32),
            pltpu.SemaphoreType.DMA,
        ],
        compiler_params=pltpu.CompilerParams(use_tc_tiling_on_sc=False),
    )
    def k(msg_hbm, dst_hbm, z_hbm, out_hbm, idx_v, rows_v, table_sh, seml):
        c = lax.axis_index("c")
        s = lax.axis_index("s")
        pltpu.sync_copy(z_hbm.at[pl.ds(s * rpt, rpt)],
                        table_sh.at[pl.ds(s * rpt, rpt)])
        plsc.subcore_barrier()
        lo = c * HALF

        def fire_load(i, b):
            pltpu.async_copy(msg_hbm.at[pl.ds(s * ept + i * CHS, CHS)],
                             rows_v.at[b], seml)
            pltpu.async_copy(dst_hbm.at[pl.ds(s * (ept // 128) + i, 1)],
                             idx_v.at[b], seml)

        def wait_load(i, b):
            pltpu.make_async_copy(msg_hbm.at[pl.ds(s * ept + i * CHS, CHS)],
                                  rows_v.at[b], seml).wait()
            pltpu.make_async_copy(dst_hbm.at[pl.ds(s * (ept // 128) + i, 1)],
                                  idx_v.at[b], seml).wait()

        def process(b):
            for q in range(0, 128, 16):
                v = idx_v[b, 0, pl.ds(q, 16)]
                loc = v - lo
                ok = (loc >= 0) & (loc < HALF)
                idx_v[b, 0, pl.ds(q, 16)] = jnp.where(ok, loc, TRASH)
            pltpu.sync_copy(rows_v.at[b], table_sh.at[idx_v.at[b, 0]],
                            add=True)

        fire_load(0, 0)

        def body2(kk, carry):
            i0 = 2 * kk
            fire_load(i0 + 1, 1)
            wait_load(i0, 0)
            process(0)

            @pl.when(kk < nchh - 1)
            def _():
                fire_load(i0 + 2, 0)

            wait_load(i0 + 1, 1)
            process(1)
            return carry

        lax.fori_loop(0, nchh, body2, 0)
        plsc.subcore_barrier()
        pltpu.sync_copy(table_sh.at[pl.ds(s * rpt, rpt)],
                        out_hbm.at[c, pl.ds(s * rpt, rpt)])

    return k(msg, dst2, zeros_tbl)


def _full(shape):
    return pl.BlockSpec(shape, lambda i: tuple(0 for _ in shape))


def _mlp_ln(xin, w1t, b1, w2t, b2, g, be, br, relu_out=False):
    n, d = xin.shape

    def body(x_ref, w1_ref, b1_ref, w2_ref, b2_ref, g_ref, be_ref, o_ref):
        h = jnp.maximum(x_ref[...] @ w1_ref[...] + b1_ref[...], 0.0)
        h = h @ w2_ref[...] + b2_ref[...]
        m = jnp.mean(h, axis=-1, keepdims=True)
        v = jnp.mean((h - m) ** 2, axis=-1, keepdims=True)
        o = (h - m) * lax.rsqrt(v + 1e-5) * g_ref[...] + be_ref[...]
        if relu_out:
            o = jnp.maximum(o, 0.0)
        o_ref[...] = o

    return pl.pallas_call(
        body,
        grid=(n // br,),
        in_specs=[
            pl.BlockSpec((br, d), lambda i: (i, 0)),
            _full((d, H)), _full((1, H)), _full((H, H)),
            _full((1, H)), _full((1, H)), _full((1, H)),
        ],
        out_specs=pl.BlockSpec((br, H), lambda i: (i, 0)),
        out_shape=jax.ShapeDtypeStruct((n, H), jnp.float32),
    )(xin, w1t, b1, w2t, b2, g, be)


def _relu_add(a, b):
    br = 4096

    def body(a_ref, b_ref, o_ref):
        o_ref[...] = jnp.maximum(a_ref[...] + b_ref[...], 0.0)

    return pl.pallas_call(
        body,
        grid=(E_PAD // br,),
        in_specs=[pl.BlockSpec((br, H), lambda i: (i, 0))] * 2,
        out_specs=pl.BlockSpec((br, H), lambda i: (i, 0)),
        out_shape=jax.ShapeDtypeStruct((E_PAD, H), jnp.float32),
    )(a, b)


def _gine_update(h, aggr, eps1, w1t, b1, w2t, b2, g, be, relu_out):
    br = 2000

    def body(h_ref, a_ref, e1_ref, w1_ref, b1_ref, w2_ref, b2_ref,
             g_ref, be_ref, o_ref):
        z = h_ref[...] * e1_ref[...] + a_ref[...]
        z = jnp.maximum(z @ w1_ref[...] + b1_ref[...], 0.0)
        z = z @ w2_ref[...] + b2_ref[...]
        m = jnp.mean(z, axis=-1, keepdims=True)
        v = jnp.mean((z - m) ** 2, axis=-1, keepdims=True)
        o = (z - m) * lax.rsqrt(v + 1e-5) * g_ref[...] + be_ref[...]
        if relu_out:
            o = jnp.maximum(o, 0.0)
        o_ref[...] = o

    return pl.pallas_call(
        body,
        grid=(N // br,),
        in_specs=[
            pl.BlockSpec((br, H), lambda i: (i, 0)),
            pl.BlockSpec((br, H), lambda i: (i, 0)),
            _full((1, H)), _full((H, H)), _full((1, H)),
            _full((H, H)), _full((1, H)), _full((1, H)), _full((1, H)),
        ],
        out_specs=pl.BlockSpec((br, H), lambda i: (i, 0)),
        out_shape=jax.ShapeDtypeStruct((N, H), jnp.float32),
    )(h, aggr, eps1, w1t, b1, w2t, b2, g, be)


def _pool_gp(h, gpwt, gpb, gpg, gpbe, wc, epb1):
    br = 2000
    steps = N // br

    def body(h_ref, gpw_ref, gpb_ref, gpg_ref, gpbe_ref, wc_ref, b1_ref,
             c0_ref, acc_ref):
        i = pl.program_id(0)

        @pl.when(i == 0)
        def _():
            acc_ref[...] = jnp.zeros_like(acc_ref)

        acc_ref[...] += jnp.sum(h_ref[...], axis=0, keepdims=True)

        @pl.when(i == steps - 1)
        def _():
            gm = acc_ref[...] * (1.0 / N)
            t = jnp.maximum(gm @ gpw_ref[...] + gpb_ref[...], 0.0)
            m = jnp.mean(t, axis=-1, keepdims=True)
            v = jnp.mean((t - m) ** 2, axis=-1, keepdims=True)
            gg = (t - m) * lax.rsqrt(v + 1e-5) * gpg_ref[...] + gpbe_ref[...]
            c0_ref[...] = gg @ wc_ref[...] + b1_ref[...]

    return pl.pallas_call(
        body,
        grid=(steps,),
        in_specs=[
            pl.BlockSpec((br, H), lambda i: (i, 0)),
            _full((H, H)), _full((1, H)), _full((1, H)), _full((1, H)),
            _full((H, 2 * H)), _full((1, 2 * H)),
        ],
        out_specs=_full((1, 2 * H)),
        out_shape=jax.ShapeDtypeStruct((1, 2 * H), jnp.float32),
        scratch_shapes=[pltpu.VMEM((1, H), jnp.float32)],
    )(h, gpwt, gpb, gpg, gpbe, wc, epb1)


def _predictor(s2, d2, e, c0, wa, wb, wd, w2t, b2, w3r, b3):
    br = 2048

    def body(s_ref, d_ref, e_ref, c0_ref, wa_ref, wb_ref, wd_ref,
             w2_ref, b2_ref, w3_ref, b3_ref, o_ref):
        z1 = (s_ref[...] @ wa_ref[...] + d_ref[...] @ wb_ref[...]
              + e_ref[...] @ wd_ref[...] + c0_ref[...])
        z1 = jnp.tanh(z1)
        z2 = jnp.tanh(z1 @ w2_ref[...] + b2_ref[...])
        sc = jnp.sum(z2 * w3_ref[...], axis=-1, keepdims=True) + b3_ref[...]
        o_ref[...] = jax.nn.sigmoid(sc)

    return pl.pallas_call(
        body,
        grid=(E_PAD // br,),
        in_specs=[
            pl.BlockSpec((br, H), lambda i: (i, 0)),
            pl.BlockSpec((br, H), lambda i: (i, 0)),
            pl.BlockSpec((br, H), lambda i: (i, 0)),
            _full((1, 2 * H)), _full((H, 2 * H)), _full((H, 2 * H)),
            _full((H, 2 * H)), _full((2 * H, H)), _full((1, H)),
            _full((1, H)), _full((1, 1)),
        ],
        out_specs=pl.BlockSpec((br, 1), lambda i: (i, 0)),
        out_shape=jax.ShapeDtypeStruct((E_PAD, 1), jnp.float32),
    )(s2, d2, e, c0, wa, wb, wd, w2t, b2, w3r, b3)


def kernel(x, edge_index, edge_attr, batch, params):
    p = params
    r1 = lambda a: a.reshape(1, -1)
    pad = E_PAD - E
    src = edge_index[0]
    dst = edge_index[1]
    src2 = jnp.concatenate([src, jnp.zeros((pad,), jnp.int32)]).reshape(E_PAD // 128, 128)
    dstg2 = jnp.concatenate([dst, jnp.zeros((pad,), jnp.int32)]).reshape(E_PAD // 128, 128)
    dsts2 = jnp.concatenate([dst, jnp.full((pad,), N, jnp.int32)]).reshape(E_PAD // 128, 128)
    ea_pad = jnp.pad(edge_attr, ((0, pad), (0, 0)))
    zeros_tbl = jnp.zeros((TBL, H), jnp.float32)

    h = _mlp_ln(x, p['ne_W1'].T, r1(p['ne_b1']), p['ne_W2'].T, r1(p['ne_b2']),
                r1(p['ne_g']), r1(p['ne_be']), br=2000)
    e = _mlp_ln(ea_pad, p['ee_W1'].T, r1(p['ee_b1']), p['ee_W2'].T, r1(p['ee_b2']),
                r1(p['ee_g']), r1(p['ee_be']), br=2048)

    for li, l in enumerate(('l0', 'l1')):
        hs = _gather_rows(h, src2, E_PAD)
        msg = _relu_add(hs, e)
        agg = _scatter_add(msg, dsts2, zeros_tbl)
        aggr = jnp.concatenate([agg[0, :HALF], agg[1, :HALF]], axis=0)
        eps1 = r1(jnp.broadcast_to(1.0 + p[l + '_eps'], (H,)))
        h = _gine_update(h, aggr, eps1, p[l + '_W1'].T, r1(p[l + '_b1']),
                         p[l + '_W2'].T, r1(p[l + '_b2']),
                         r1(p[l + '_g']), r1(p[l + '_be']), relu_out=(li == 0))

    w1t = p['ep_W1'].T
    c0 = _pool_gp(h, p['gp_W'].T, r1(p['gp_b']), r1(p['gp_g']), r1(p['gp_be']),
                  w1t[2 * H:3 * H], r1(p['ep_b1']))
    sd = _gather_rows(h, jnp.concatenate([src2, dstg2]), 2 * E_PAD)
    s2 = sd[:E_PAD]
    d2 = sd[E_PAD:]
    out = _predictor(s2, d2, e, c0, w1t[:H], w1t[H:2 * H], w1t[3 * H:],
                     p['ep_W2'].T, r1(p['ep_b2']), r1(p['ep_W3']), r1(p['ep_b3']))
    return out[:E]


# speedup vs baseline: 1.1713x; 1.1713x over previous
import functools

import jax
import jax.numpy as jnp
from jax import lax
from jax.experimental import pallas as pl
from jax.experimental.pallas import tpu as pltpu
from jax.experimental.pallas import tpu_sc as plsc

N = 50000
E = 800000
H = 64
E_PAD = 819200
NC, NS = 2, 16
NW = NC * NS
HALF = N // 2
TBL = 25024
TRASH = TBL - 8
CH = 512
CHS = 128


def _gather_rows(table, idx2, n_rows):
    epw = n_rows // NW
    nch = epw // CH
    nchh = nch // 2
    jj = CH // 128
    mesh = plsc.VectorSubcoreMesh(core_axis_name="c", subcore_axis_name="s")

    @functools.partial(
        pl.kernel,
        mesh=mesh,
        out_type=jax.ShapeDtypeStruct((n_rows, H), jnp.float32),
        scratch_types=[
            pltpu.VMEM((2, jj, 128), jnp.int32),
            pltpu.VMEM((2, CH, H), jnp.float32),
            pltpu.SemaphoreType.DMA,
        ],
        compiler_params=pltpu.CompilerParams(use_tc_tiling_on_sc=False),
    )
    def k(table_hbm, idx_hbm, out_hbm, idx_v, rows_v, semg):
        c = lax.axis_index("c")
        s = lax.axis_index("s")
        wid = s * NC + c
        base = wid * epw
        irow = wid * (epw // 128)

        def load_idx(i, b):
            pltpu.sync_copy(idx_hbm.at[pl.ds(irow + i * jj, jj)], idx_v.at[b])

        def fire(b):
            for j in range(jj):
                pltpu.async_copy(table_hbm.at[idx_v.at[b, j]],
                                 rows_v.at[b, pl.ds(j * 128, 128)], semg)

        def drain(b):
            for j in range(jj):
                pltpu.make_async_copy(table_hbm.at[idx_v.at[b, j]],
                                      rows_v.at[b, pl.ds(j * 128, 128)],
                                      semg).wait()

        def store(i, b):
            pltpu.sync_copy(rows_v.at[b], out_hbm.at[pl.ds(base + i * CH, CH)])

        load_idx(0, 0)
        fire(0)

        def body2(kk, carry):
            i0 = 2 * kk
            load_idx(i0 + 1, 1)
            drain(0)
            fire(1)
            store(i0, 0)

            @pl.when(kk < nchh - 1)
            def _():
                load_idx(i0 + 2, 0)
                fire(0)

            drain(1)
            store(i0 + 1, 1)
            return carry

        lax.fori_loop(0, nchh, body2, 0)

    return k(table, idx2)


def _gather_msg(table, e, idx2):
    epw = E_PAD // NW
    nch = epw // CH
    nchh = nch // 2
    jj = CH // 128
    mesh = plsc.VectorSubcoreMesh(core_axis_name="c", subcore_axis_name="s")

    @functools.partial(
        pl.kernel,
        mesh=mesh,
        out_type=jax.ShapeDtypeStruct((E_PAD, H), jnp.float32),
        scratch_types=[
            pltpu.VMEM((2, jj, 128), jnp.int32),
            pltpu.VMEM((2, CH, H), jnp.float32),
            pltpu.VMEM((CH, H), jnp.float32),
            pltpu.SemaphoreType.DMA,
            pltpu.SemaphoreType.DMA,
        ],
        compiler_params=pltpu.CompilerParams(use_tc_tiling_on_sc=False),
    )
    def k(table_hbm, e_hbm, idx_hbm, out_hbm, idx_v, rows_v, e_v, semg, seme):
        c = lax.axis_index("c")
        s = lax.axis_index("s")
        wid = c * NS + s
        base = wid * epw
        irow = wid * (epw // 128)

        def load_idx(i, b):
            pltpu.sync_copy(idx_hbm.at[pl.ds(irow + i * jj, jj)], idx_v.at[b])

        def fire_h(b):
            for j in range(jj):
                pltpu.async_copy(table_hbm.at[idx_v.at[b, j]],
                                 rows_v.at[b, pl.ds(j * 128, 128)], semg)

        def drain_h(b):
            for j in range(jj):
                pltpu.make_async_copy(table_hbm.at[idx_v.at[b, j]],
                                      rows_v.at[b, pl.ds(j * 128, 128)],
                                      semg).wait()

        def fire_e(i):
            pltpu.async_copy(e_hbm.at[pl.ds(base + i * CH, CH)], e_v, seme)

        def drain_e(i):
            pltpu.make_async_copy(e_hbm.at[pl.ds(base + i * CH, CH)], e_v,
                                  seme).wait()

        def relu_add(b):
            def row2(r, carry):
                r0 = 2 * r
                for dr in range(2):
                    for q in range(0, H, 16):
                        hv = rows_v[b, r0 + dr, pl.ds(q, 16)]
                        ev = e_v[r0 + dr, pl.ds(q, 16)]
                        rows_v[b, r0 + dr, pl.ds(q, 16)] = (
                            jnp.maximum(hv + ev, 0.0))
                return carry

            lax.fori_loop(0, CH // 2, row2, 0)

        def store(i, b):
            pltpu.sync_copy(rows_v.at[b], out_hbm.at[pl.ds(base + i * CH, CH)])

        load_idx(0, 0)
        fire_h(0)
        fire_e(0)

        def body2(kk, carry):
            i0 = 2 * kk
            load_idx(i0 + 1, 1)
            fire_h(1)
            drain_h(0)
            drain_e(i0)
            relu_add(0)
            fire_e(i0 + 1)
            store(i0, 0)

            @pl.when(kk < nchh - 1)
            def _():
                load_idx(i0 + 2, 0)
                fire_h(0)

            drain_h(1)
            drain_e(i0 + 1)
            relu_add(1)

            @pl.when(kk < nchh - 1)
            def _():
                fire_e(i0 + 2)

            store(i0 + 1, 1)
            return carry

        lax.fori_loop(0, nchh, body2, 0)

    return k(table, e, idx2)


def _scatter_add(msg, dst2, zeros_tbl):
    ept = E_PAD // NS
    nch = ept // CHS
    nchh = nch // 2
    rpt = TBL // NS
    mesh = plsc.VectorSubcoreMesh(core_axis_name="c", subcore_axis_name="s")

    @functools.partial(
        pl.kernel,
        mesh=mesh,
        out_type=jax.ShapeDtypeStruct((NC, TBL, H), jnp.float32),
        scratch_types=[
            pltpu.VMEM((2, 1, 128), jnp.int32),
            pltpu.VMEM((2, CHS, H), jnp.float32),
            pltpu.VMEM_SHARED((TBL, H), jnp.float32),
            pltpu.SemaphoreType.DMA,
        ],
        compiler_params=pltpu.CompilerParams(use_tc_tiling_on_sc=False),
    )
    def k(msg_hbm, dst_hbm, z_hbm, out_hbm, idx_v, rows_v, table_sh, seml):
        c = lax.axis_index("c")
        s = lax.axis_index("s")
        pltpu.sync_copy(z_hbm.at[pl.ds(s * rpt, rpt)],
                        table_sh.at[pl.ds(s * rpt, rpt)])
        plsc.subcore_barrier()
        lo = c * HALF

        def fire_load(i, b):
            pltpu.async_copy(msg_hbm.at[pl.ds(s * ept + i * CHS, CHS)],
                             rows_v.at[b], seml)
            pltpu.async_copy(dst_hbm.at[pl.ds(s * (ept // 128) + i, 1)],
                             idx_v.at[b], seml)

        def wait_load(i, b):
            pltpu.make_async_copy(msg_hbm.at[pl.ds(s * ept + i * CHS, CHS)],
                                  rows_v.at[b], seml).wait()
            pltpu.make_async_copy(dst_hbm.at[pl.ds(s * (ept // 128) + i, 1)],
                                  idx_v.at[b], seml).wait()

        def process(b):
            for q in range(0, 128, 16):
                v = idx_v[b, 0, pl.ds(q, 16)]
                loc = v - lo
                ok = (loc >= 0) & (loc < HALF)
                idx_v[b, 0, pl.ds(q, 16)] = jnp.where(ok, loc, TRASH)
            pltpu.sync_copy(rows_v.at[b], table_sh.at[idx_v.at[b, 0]],
                            add=True)

        fire_load(0, 0)

        def body2(kk, carry):
            i0 = 2 * kk
            fire_load(i0 + 1, 1)
            wait_load(i0, 0)
            process(0)

            @pl.when(kk < nchh - 1)
            def _():
                fire_load(i0 + 2, 0)

            wait_load(i0 + 1, 1)
            process(1)
            return carry

        lax.fori_loop(0, nchh, body2, 0)
        plsc.subcore_barrier()
        pltpu.sync_copy(table_sh.at[pl.ds(s * rpt, rpt)],
                        out_hbm.at[c, pl.ds(s * rpt, rpt)])

    return k(msg, dst2, zeros_tbl)


def _full(shape):
    return pl.BlockSpec(shape, lambda i: tuple(0 for _ in shape))


def _mlp_ln(xin, w1t, b1, w2t, b2, g, be, br, relu_out=False):
    n, d = xin.shape

    def body(x_ref, w1_ref, b1_ref, w2_ref, b2_ref, g_ref, be_ref, o_ref):
        h = jnp.maximum(x_ref[...] @ w1_ref[...] + b1_ref[...], 0.0)
        h = h @ w2_ref[...] + b2_ref[...]
        m = jnp.mean(h, axis=-1, keepdims=True)
        v = jnp.mean((h - m) ** 2, axis=-1, keepdims=True)
        o = (h - m) * lax.rsqrt(v + 1e-5) * g_ref[...] + be_ref[...]
        if relu_out:
            o = jnp.maximum(o, 0.0)
        o_ref[...] = o

    return pl.pallas_call(
        body,
        grid=(n // br,),
        in_specs=[
            pl.BlockSpec((br, d), lambda i: (i, 0)),
            _full((d, H)), _full((1, H)), _full((H, H)),
            _full((1, H)), _full((1, H)), _full((1, H)),
        ],
        out_specs=pl.BlockSpec((br, H), lambda i: (i, 0)),
        out_shape=jax.ShapeDtypeStruct((n, H), jnp.float32),
    )(xin, w1t, b1, w2t, b2, g, be)


def _gine_update(h, aggr, eps1, w1t, b1, w2t, b2, g, be, relu_out):
    br = 2000

    def body(h_ref, a_ref, e1_ref, w1_ref, b1_ref, w2_ref, b2_ref,
             g_ref, be_ref, o_ref):
        z = h_ref[...] * e1_ref[...] + a_ref[...]
        z = jnp.maximum(z @ w1_ref[...] + b1_ref[...], 0.0)
        z = z @ w2_ref[...] + b2_ref[...]
        m = jnp.mean(z, axis=-1, keepdims=True)
        v = jnp.mean((z - m) ** 2, axis=-1, keepdims=True)
        o = (z - m) * lax.rsqrt(v + 1e-5) * g_ref[...] + be_ref[...]
        if relu_out:
            o = jnp.maximum(o, 0.0)
        o_ref[...] = o

    return pl.pallas_call(
        body,
        grid=(N // br,),
        in_specs=[
            pl.BlockSpec((br, H), lambda i: (i, 0)),
            pl.BlockSpec((br, H), lambda i: (i, 0)),
            _full((1, H)), _full((H, H)), _full((1, H)),
            _full((H, H)), _full((1, H)), _full((1, H)), _full((1, H)),
        ],
        out_specs=pl.BlockSpec((br, H), lambda i: (i, 0)),
        out_shape=jax.ShapeDtypeStruct((N, H), jnp.float32),
    )(h, aggr, eps1, w1t, b1, w2t, b2, g, be)


def _pool_gp(h, gpwt, gpb, gpg, gpbe, wc, epb1):
    br = 2000
    steps = N // br

    def body(h_ref, gpw_ref, gpb_ref, gpg_ref, gpbe_ref, wc_ref, b1_ref,
             c0_ref, acc_ref):
        i = pl.program_id(0)

        @pl.when(i == 0)
        def _():
            acc_ref[...] = jnp.zeros_like(acc_ref)

        acc_ref[...] += jnp.sum(h_ref[...], axis=0, keepdims=True)

        @pl.when(i == steps - 1)
        def _():
            gm = acc_ref[...] * (1.0 / N)
            t = jnp.maximum(gm @ gpw_ref[...] + gpb_ref[...], 0.0)
            m = jnp.mean(t, axis=-1, keepdims=True)
            v = jnp.mean((t - m) ** 2, axis=-1, keepdims=True)
            gg = (t - m) * lax.rsqrt(v + 1e-5) * gpg_ref[...] + gpbe_ref[...]
            c0_ref[...] = gg @ wc_ref[...] + b1_ref[...]

    return pl.pallas_call(
        body,
        grid=(steps,),
        in_specs=[
            pl.BlockSpec((br, H), lambda i: (i, 0)),
            _full((H, H)), _full((1, H)), _full((1, H)), _full((1, H)),
            _full((H, 2 * H)), _full((1, 2 * H)),
        ],
        out_specs=_full((1, 2 * H)),
        out_shape=jax.ShapeDtypeStruct((1, 2 * H), jnp.float32),
        scratch_shapes=[pltpu.VMEM((1, H), jnp.float32)],
    )(h, gpwt, gpb, gpg, gpbe, wc, epb1)


def _predictor(s2, d2, e, c0, wa, wb, wd, w2t, b2, w3r, b3):
    br = 2048

    def body(s_ref, d_ref, e_ref, c0_ref, wa_ref, wb_ref, wd_ref,
             w2_ref, b2_ref, w3_ref, b3_ref, o_ref):
        z1 = (s_ref[...] @ wa_ref[...] + d_ref[...] @ wb_ref[...]
              + e_ref[...] @ wd_ref[...] + c0_ref[...])
        z1 = jnp.tanh(z1)
        z2 = jnp.tanh(z1 @ w2_ref[...] + b2_ref[...])
        sc = jnp.sum(z2 * w3_ref[...], axis=-1, keepdims=True) + b3_ref[...]
        o_ref[...] = jax.nn.sigmoid(sc)

    return pl.pallas_call(
        body,
        grid=(E_PAD // br,),
        in_specs=[
            pl.BlockSpec((br, H), lambda i: (i, 0)),
            pl.BlockSpec((br, H), lambda i: (i, 0)),
            pl.BlockSpec((br, H), lambda i: (i, 0)),
            _full((1, 2 * H)), _full((H, 2 * H)), _full((H, 2 * H)),
            _full((H, 2 * H)), _full((2 * H, H)), _full((1, H)),
            _full((1, H)), _full((1, 1)),
        ],
        out_specs=pl.BlockSpec((br, 1), lambda i: (i, 0)),
        out_shape=jax.ShapeDtypeStruct((E_PAD, 1), jnp.float32),
    )(s2, d2, e, c0, wa, wb, wd, w2t, b2, w3r, b3)


def kernel(x, edge_index, edge_attr, batch, params):
    p = params
    r1 = lambda a: a.reshape(1, -1)
    pad = E_PAD - E
    src = edge_index[0]
    dst = edge_index[1]
    src2 = jnp.concatenate([src, jnp.zeros((pad,), jnp.int32)]).reshape(E_PAD // 128, 128)
    dstg2 = jnp.concatenate([dst, jnp.zeros((pad,), jnp.int32)]).reshape(E_PAD // 128, 128)
    dsts2 = jnp.concatenate([dst, jnp.full((pad,), N, jnp.int32)]).reshape(E_PAD // 128, 128)
    ea_pad = jnp.pad(edge_attr, ((0, pad), (0, 0)))
    zeros_tbl = jnp.zeros((TBL, H), jnp.float32)

    h = _mlp_ln(x, p['ne_W1'].T, r1(p['ne_b1']), p['ne_W2'].T, r1(p['ne_b2']),
                r1(p['ne_g']), r1(p['ne_be']), br=2000)
    e = _mlp_ln(ea_pad, p['ee_W1'].T, r1(p['ee_b1']), p['ee_W2'].T, r1(p['ee_b2']),
                r1(p['ee_g']), r1(p['ee_be']), br=2048)

    for li, l in enumerate(('l0', 'l1')):
        msg = _gather_msg(h, e, src2)
        agg = _scatter_add(msg, dsts2, zeros_tbl)
        aggr = jnp.concatenate([agg[0, :HALF], agg[1, :HALF]], axis=0)
        eps1 = r1(jnp.broadcast_to(1.0 + p[l + '_eps'], (H,)))
        h = _gine_update(h, aggr, eps1, p[l + '_W1'].T, r1(p[l + '_b1']),
                         p[l + '_W2'].T, r1(p[l + '_b2']),
                         r1(p[l + '_g']), r1(p[l + '_be']), relu_out=(li == 0))

    w1t = p['ep_W1'].T
    c0 = _pool_gp(h, p['gp_W'].T, r1(p['gp_b']), r1(p['gp_g']), r1(p['gp_be']),
                  w1t[2 * H:3 * H], r1(p['ep_b1']))
    sd = _gather_rows(h, jnp.concatenate([src2, dstg2]), 2 * E_PAD)
    s2 = sd[:E_PAD]
    d2 = sd[E_PAD:]
    out = _predictor(s2, d2, e, c0, w1t[:H], w1t[H:2 * H], w1t[3 * H:],
                     p['ep_W2'].T, r1(p['ep_b2']), r1(p['ep_W3']), r1(p['ep_b3']))
    return out[:E]


# speedup vs baseline: 1.2371x; 1.0562x over previous
import functools

import jax
import jax.numpy as jnp
from jax import lax
from jax.experimental import pallas as pl
from jax.experimental.pallas import tpu as pltpu
from jax.experimental.pallas import tpu_sc as plsc

N = 50000
E = 800000
H = 64
E_PAD = 819200
NC, NS = 2, 16
NW = NC * NS
HALF = N // 2
TBL = 25024
TRASH = TBL - 8
CH = 512
CHS = 128


def _gather_rows(table, idx2, n_rows):
    epw = n_rows // NW
    nch = epw // CH
    nchh = nch // 2
    jj = CH // 128
    dt = table.dtype
    mesh = plsc.VectorSubcoreMesh(core_axis_name="c", subcore_axis_name="s")

    @functools.partial(
        pl.kernel,
        mesh=mesh,
        out_type=jax.ShapeDtypeStruct((n_rows, H), dt),
        scratch_types=[
            pltpu.VMEM((2, jj, 128), jnp.int32),
            pltpu.VMEM((2, CH, H), dt),
            pltpu.SemaphoreType.DMA,
        ],
        compiler_params=pltpu.CompilerParams(use_tc_tiling_on_sc=False),
    )
    def k(table_hbm, idx_hbm, out_hbm, idx_v, rows_v, semg):
        c = lax.axis_index("c")
        s = lax.axis_index("s")
        wid = s * NC + c
        base = wid * epw
        irow = wid * (epw // 128)

        def load_idx(i, b):
            pltpu.sync_copy(idx_hbm.at[pl.ds(irow + i * jj, jj)], idx_v.at[b])

        def fire(b):
            for j in range(jj):
                pltpu.async_copy(table_hbm.at[idx_v.at[b, j]],
                                 rows_v.at[b, pl.ds(j * 128, 128)], semg)

        def drain(b):
            for j in range(jj):
                pltpu.make_async_copy(table_hbm.at[idx_v.at[b, j]],
                                      rows_v.at[b, pl.ds(j * 128, 128)],
                                      semg).wait()

        def store(i, b):
            pltpu.sync_copy(rows_v.at[b], out_hbm.at[pl.ds(base + i * CH, CH)])

        load_idx(0, 0)
        fire(0)

        def body2(kk, carry):
            i0 = 2 * kk
            load_idx(i0 + 1, 1)
            drain(0)
            fire(1)
            store(i0, 0)

            @pl.when(kk < nchh - 1)
            def _():
                load_idx(i0 + 2, 0)
                fire(0)

            drain(1)
            store(i0 + 1, 1)
            return carry

        lax.fori_loop(0, nchh, body2, 0)

    return k(table, idx2)


def _gather_msg(table, e, idx2):
    epw = E_PAD // NW
    nch = epw // CH
    nchh = nch // 2
    jj = CH // 128
    mesh = plsc.VectorSubcoreMesh(core_axis_name="c", subcore_axis_name="s")

    @functools.partial(
        pl.kernel,
        mesh=mesh,
        out_type=jax.ShapeDtypeStruct((E_PAD, H), jnp.float32),
        scratch_types=[
            pltpu.VMEM((2, jj, 128), jnp.int32),
            pltpu.VMEM((2, CH, H), jnp.float32),
            pltpu.VMEM((CH, H), jnp.float32),
            pltpu.SemaphoreType.DMA,
            pltpu.SemaphoreType.DMA,
        ],
        compiler_params=pltpu.CompilerParams(use_tc_tiling_on_sc=False),
    )
    def k(table_hbm, e_hbm, idx_hbm, out_hbm, idx_v, rows_v, e_v, semg, seme):
        c = lax.axis_index("c")
        s = lax.axis_index("s")
        wid = c * NS + s
        base = wid * epw
        irow = wid * (epw // 128)

        def load_idx(i, b):
            pltpu.sync_copy(idx_hbm.at[pl.ds(irow + i * jj, jj)], idx_v.at[b])

        def fire_h(b):
            for j in range(jj):
                pltpu.async_copy(table_hbm.at[idx_v.at[b, j]],
                                 rows_v.at[b, pl.ds(j * 128, 128)], semg)

        def drain_h(b):
            for j in range(jj):
                pltpu.make_async_copy(table_hbm.at[idx_v.at[b, j]],
                                      rows_v.at[b, pl.ds(j * 128, 128)],
                                      semg).wait()

        def fire_e(i):
            pltpu.async_copy(e_hbm.at[pl.ds(base + i * CH, CH)], e_v, seme)

        def drain_e(i):
            pltpu.make_async_copy(e_hbm.at[pl.ds(base + i * CH, CH)], e_v,
                                  seme).wait()

        def relu_add(b):
            def row2(r, carry):
                r0 = 2 * r
                for dr in range(2):
                    for q in range(0, H, 16):
                        hv = rows_v[b, r0 + dr, pl.ds(q, 16)]
                        ev = e_v[r0 + dr, pl.ds(q, 16)]
                        rows_v[b, r0 + dr, pl.ds(q, 16)] = (
                            jnp.maximum(hv + ev, 0.0))
                return carry

            lax.fori_loop(0, CH // 2, row2, 0)

        def store(i, b):
            pltpu.sync_copy(rows_v.at[b], out_hbm.at[pl.ds(base + i * CH, CH)])

        load_idx(0, 0)
        fire_h(0)
        fire_e(0)

        def body2(kk, carry):
            i0 = 2 * kk
            load_idx(i0 + 1, 1)
            fire_h(1)
            drain_h(0)
            drain_e(i0)
            relu_add(0)
            fire_e(i0 + 1)
            store(i0, 0)

            @pl.when(kk < nchh - 1)
            def _():
                load_idx(i0 + 2, 0)
                fire_h(0)

            drain_h(1)
            drain_e(i0 + 1)
            relu_add(1)

            @pl.when(kk < nchh - 1)
            def _():
                fire_e(i0 + 2)

            store(i0 + 1, 1)
            return carry

        lax.fori_loop(0, nchh, body2, 0)

    return k(table, e, idx2)


def _scatter_add(msg, dst2, zeros_tbl):
    ept = E_PAD // NS
    nch = ept // CHS
    nchh = nch // 2
    rpt = TBL // NS
    mesh = plsc.VectorSubcoreMesh(core_axis_name="c", subcore_axis_name="s")

    @functools.partial(
        pl.kernel,
        mesh=mesh,
        out_type=jax.ShapeDtypeStruct((NC, TBL, H), jnp.float32),
        scratch_types=[
            pltpu.VMEM((2, 1, 128), jnp.int32),
            pltpu.VMEM((2, CHS, H), jnp.float32),
            pltpu.VMEM_SHARED((TBL, H), jnp.float32),
            pltpu.SemaphoreType.DMA,
        ],
        compiler_params=pltpu.CompilerParams(use_tc_tiling_on_sc=False),
    )
    def k(msg_hbm, dst_hbm, z_hbm, out_hbm, idx_v, rows_v, table_sh, seml):
        c = lax.axis_index("c")
        s = lax.axis_index("s")
        pltpu.sync_copy(z_hbm.at[pl.ds(s * rpt, rpt)],
                        table_sh.at[pl.ds(s * rpt, rpt)])
        plsc.subcore_barrier()
        lo = c * HALF

        def fire_load(i, b):
            pltpu.async_copy(msg_hbm.at[pl.ds(s * ept + i * CHS, CHS)],
                             rows_v.at[b], seml)
            pltpu.async_copy(dst_hbm.at[pl.ds(s * (ept // 128) + i, 1)],
                             idx_v.at[b], seml)

        def wait_load(i, b):
            pltpu.make_async_copy(msg_hbm.at[pl.ds(s * ept + i * CHS, CHS)],
                                  rows_v.at[b], seml).wait()
            pltpu.make_async_copy(dst_hbm.at[pl.ds(s * (ept // 128) + i, 1)],
                                  idx_v.at[b], seml).wait()

        def process(b):
            for q in range(0, 128, 16):
                v = idx_v[b, 0, pl.ds(q, 16)]
                loc = v - lo
                ok = (loc >= 0) & (loc < HALF)
                idx_v[b, 0, pl.ds(q, 16)] = jnp.where(ok, loc, TRASH)
            pltpu.sync_copy(rows_v.at[b], table_sh.at[idx_v.at[b, 0]],
                            add=True)

        fire_load(0, 0)

        def body2(kk, carry):
            i0 = 2 * kk
            fire_load(i0 + 1, 1)
            wait_load(i0, 0)
            process(0)

            @pl.when(kk < nchh - 1)
            def _():
                fire_load(i0 + 2, 0)

            wait_load(i0 + 1, 1)
            process(1)
            return carry

        lax.fori_loop(0, nchh, body2, 0)
        plsc.subcore_barrier()
        pltpu.sync_copy(table_sh.at[pl.ds(s * rpt, rpt)],
                        out_hbm.at[c, pl.ds(s * rpt, rpt)])

    return k(msg, dst2, zeros_tbl)


def _full(shape):
    return pl.BlockSpec(shape, lambda i: tuple(0 for _ in shape))


def _mlp_ln(xin, w1t, b1, w2t, b2, g, be, br, relu_out=False):
    n, d = xin.shape

    def body(x_ref, w1_ref, b1_ref, w2_ref, b2_ref, g_ref, be_ref, o_ref):
        h = jnp.maximum(x_ref[...] @ w1_ref[...] + b1_ref[...], 0.0)
        h = h @ w2_ref[...] + b2_ref[...]
        m = jnp.mean(h, axis=-1, keepdims=True)
        v = jnp.mean((h - m) ** 2, axis=-1, keepdims=True)
        o = (h - m) * lax.rsqrt(v + 1e-5) * g_ref[...] + be_ref[...]
        if relu_out:
            o = jnp.maximum(o, 0.0)
        o_ref[...] = o

    return pl.pallas_call(
        body,
        grid=(n // br,),
        in_specs=[
            pl.BlockSpec((br, d), lambda i: (i, 0)),
            _full((d, H)), _full((1, H)), _full((H, H)),
            _full((1, H)), _full((1, H)), _full((1, H)),
        ],
        out_specs=pl.BlockSpec((br, H), lambda i: (i, 0)),
        out_shape=jax.ShapeDtypeStruct((n, H), jnp.float32),
    )(xin, w1t, b1, w2t, b2, g, be)


def _gine_update(h, aggr, eps1, w1t, b1, w2t, b2, g, be, relu_out):
    br = 2000

    def body(h_ref, a_ref, e1_ref, w1_ref, b1_ref, w2_ref, b2_ref,
             g_ref, be_ref, o_ref):
        z = h_ref[...] * e1_ref[...] + a_ref[...]
        z = jnp.maximum(z @ w1_ref[...] + b1_ref[...], 0.0)
        z = z @ w2_ref[...] + b2_ref[...]
        m = jnp.mean(z, axis=-1, keepdims=True)
        v = jnp.mean((z - m) ** 2, axis=-1, keepdims=True)
        o = (z - m) * lax.rsqrt(v + 1e-5) * g_ref[...] + be_ref[...]
        if relu_out:
            o = jnp.maximum(o, 0.0)
        o_ref[...] = o

    return pl.pallas_call(
        body,
        grid=(N // br,),
        in_specs=[
            pl.BlockSpec((br, H), lambda i: (i, 0)),
            pl.BlockSpec((br, H), lambda i: (i, 0)),
            _full((1, H)), _full((H, H)), _full((1, H)),
            _full((H, H)), _full((1, H)), _full((1, H)), _full((1, H)),
        ],
        out_specs=pl.BlockSpec((br, H), lambda i: (i, 0)),
        out_shape=jax.ShapeDtypeStruct((N, H), jnp.float32),
    )(h, aggr, eps1, w1t, b1, w2t, b2, g, be)


def _pool_gp(h, gpwt, gpb, gpg, gpbe, wc, epb1):
    br = 2000
    steps = N // br

    def body(h_ref, gpw_ref, gpb_ref, gpg_ref, gpbe_ref, wc_ref, b1_ref,
             c0_ref, acc_ref):
        i = pl.program_id(0)

        @pl.when(i == 0)
        def _():
            acc_ref[...] = jnp.zeros_like(acc_ref)

        acc_ref[...] += jnp.sum(h_ref[...], axis=0, keepdims=True)

        @pl.when(i == steps - 1)
        def _():
            gm = acc_ref[...] * (1.0 / N)
            t = jnp.maximum(gm @ gpw_ref[...] + gpb_ref[...], 0.0)
            m = jnp.mean(t, axis=-1, keepdims=True)
            v = jnp.mean((t - m) ** 2, axis=-1, keepdims=True)
            gg = (t - m) * lax.rsqrt(v + 1e-5) * gpg_ref[...] + gpbe_ref[...]
            c0_ref[...] = gg @ wc_ref[...] + b1_ref[...]

    return pl.pallas_call(
        body,
        grid=(steps,),
        in_specs=[
            pl.BlockSpec((br, H), lambda i: (i, 0)),
            _full((H, H)), _full((1, H)), _full((1, H)), _full((1, H)),
            _full((H, 2 * H)), _full((1, 2 * H)),
        ],
        out_specs=_full((1, 2 * H)),
        out_shape=jax.ShapeDtypeStruct((1, 2 * H), jnp.float32),
        scratch_shapes=[pltpu.VMEM((1, H), jnp.float32)],
    )(h, gpwt, gpb, gpg, gpbe, wc, epb1)


def _predictor(s2, d2, e, c0, wa, wb, wd, w2t, b2, w3r, b3):
    br = 2048

    def body(s_ref, d_ref, e_ref, c0_ref, wa_ref, wb_ref, wd_ref,
             w2_ref, b2_ref, w3_ref, b3_ref, o_ref):
        sf = s_ref[...].astype(jnp.float32)
        df = d_ref[...].astype(jnp.float32)
        z1 = (sf @ wa_ref[...] + df @ wb_ref[...]
              + e_ref[...] @ wd_ref[...] + c0_ref[...])
        z1 = jnp.tanh(z1)
        z2 = jnp.tanh(z1 @ w2_ref[...] + b2_ref[...])
        sc = jnp.sum(z2 * w3_ref[...], axis=-1, keepdims=True) + b3_ref[...]
        o_ref[...] = jax.nn.sigmoid(sc)

    return pl.pallas_call(
        body,
        grid=(E_PAD // br,),
        in_specs=[
            pl.BlockSpec((br, H), lambda i: (i, 0)),
            pl.BlockSpec((br, H), lambda i: (i, 0)),
            pl.BlockSpec((br, H), lambda i: (i, 0)),
            _full((1, 2 * H)), _full((H, 2 * H)), _full((H, 2 * H)),
            _full((H, 2 * H)), _full((2 * H, H)), _full((1, H)),
            _full((1, H)), _full((1, 1)),
        ],
        out_specs=pl.BlockSpec((br, 1), lambda i: (i, 0)),
        out_shape=jax.ShapeDtypeStruct((E_PAD, 1), jnp.float32),
    )(s2, d2, e, c0, wa, wb, wd, w2t, b2, w3r, b3)


def kernel(x, edge_index, edge_attr, batch, params):
    p = params
    r1 = lambda a: a.reshape(1, -1)
    pad = E_PAD - E
    src = edge_index[0]
    dst = edge_index[1]
    src2 = jnp.concatenate([src, jnp.zeros((pad,), jnp.int32)]).reshape(E_PAD // 128, 128)
    dstg2 = jnp.concatenate([dst, jnp.zeros((pad,), jnp.int32)]).reshape(E_PAD // 128, 128)
    dsts2 = jnp.concatenate([dst, jnp.full((pad,), N, jnp.int32)]).reshape(E_PAD // 128, 128)
    ea_pad = jnp.pad(edge_attr, ((0, pad), (0, 0)))
    zeros_tbl = jnp.zeros((TBL, H), jnp.float32)

    h = _mlp_ln(x, p['ne_W1'].T, r1(p['ne_b1']), p['ne_W2'].T, r1(p['ne_b2']),
                r1(p['ne_g']), r1(p['ne_be']), br=2000)
    e = _mlp_ln(ea_pad, p['ee_W1'].T, r1(p['ee_b1']), p['ee_W2'].T, r1(p['ee_b2']),
                r1(p['ee_g']), r1(p['ee_be']), br=2048)

    for li, l in enumerate(('l0', 'l1')):
        msg = _gather_msg(h, e, src2)
        agg = _scatter_add(msg, dsts2, zeros_tbl)
        aggr = jnp.concatenate([agg[0, :HALF], agg[1, :HALF]], axis=0)
        eps1 = r1(jnp.broadcast_to(1.0 + p[l + '_eps'], (H,)))
        h = _gine_update(h, aggr, eps1, p[l + '_W1'].T, r1(p[l + '_b1']),
                         p[l + '_W2'].T, r1(p[l + '_b2']),
                         r1(p[l + '_g']), r1(p[l + '_be']), relu_out=(li == 0))

    w1t = p['ep_W1'].T
    c0 = _pool_gp(h, p['gp_W'].T, r1(p['gp_b']), r1(p['gp_g']), r1(p['gp_be']),
                  w1t[2 * H:3 * H], r1(p['ep_b1']))
    sd = _gather_rows(h.astype(jnp.bfloat16),
                      jnp.concatenate([src2, dstg2]), 2 * E_PAD)
    s2 = sd[:E_PAD]
    d2 = sd[E_PAD:]
    out = _predictor(s2, d2, e, c0, w1t[:H], w1t[H:2 * H], w1t[3 * H:],
                     p['ep_W2'].T, r1(p['ep_b2']), r1(p['ep_W3']), r1(p['ep_b3']))
    return out[:E]


# speedup vs baseline: 1.3750x; 1.1115x over previous
import functools

import jax
import jax.numpy as jnp
from jax import lax
from jax.experimental import pallas as pl
from jax.experimental.pallas import tpu as pltpu
from jax.experimental.pallas import tpu_sc as plsc

N = 50000
E = 800000
H = 64
E_PAD = 819200
NC, NS = 2, 16
NW = NC * NS
HH = H // 2
TBL = 50016
CH = 512
CHS = 256


def _gather_rows(table, idx2, n_rows):
    epw = n_rows // NW
    nch = epw // CH
    nchh = nch // 2
    jj = CH // 128
    dt = table.dtype
    mesh = plsc.VectorSubcoreMesh(core_axis_name="c", subcore_axis_name="s")

    @functools.partial(
        pl.kernel,
        mesh=mesh,
        out_type=jax.ShapeDtypeStruct((n_rows, H), dt),
        scratch_types=[
            pltpu.VMEM((2, jj, 128), jnp.int32),
            pltpu.VMEM((2, CH, H), dt),
            pltpu.SemaphoreType.DMA,
        ],
        compiler_params=pltpu.CompilerParams(use_tc_tiling_on_sc=False),
    )
    def k(table_hbm, idx_hbm, out_hbm, idx_v, rows_v, semg):
        c = lax.axis_index("c")
        s = lax.axis_index("s")
        wid = s * NC + c
        base = wid * epw
        irow = wid * (epw // 128)

        def load_idx(i, b):
            pltpu.sync_copy(idx_hbm.at[pl.ds(irow + i * jj, jj)], idx_v.at[b])

        def fire(b):
            for j in range(jj):
                pltpu.async_copy(table_hbm.at[idx_v.at[b, j]],
                                 rows_v.at[b, pl.ds(j * 128, 128)], semg)

        def drain(b):
            for j in range(jj):
                pltpu.make_async_copy(table_hbm.at[idx_v.at[b, j]],
                                      rows_v.at[b, pl.ds(j * 128, 128)],
                                      semg).wait()

        def store(i, b):
            pltpu.sync_copy(rows_v.at[b], out_hbm.at[pl.ds(base + i * CH, CH)])

        load_idx(0, 0)
        fire(0)

        def body2(kk, carry):
            i0 = 2 * kk
            load_idx(i0 + 1, 1)
            drain(0)
            fire(1)
            store(i0, 0)

            @pl.when(kk < nchh - 1)
            def _():
                load_idx(i0 + 2, 0)
                fire(0)

            drain(1)
            store(i0 + 1, 1)
            return carry

        lax.fori_loop(0, nchh, body2, 0)

    return k(table, idx2)


def _gather_msg(table, e, idx2):
    epw = E_PAD // NW
    nch = epw // CH
    nchh = nch // 2
    jj = CH // 128
    mesh = plsc.VectorSubcoreMesh(core_axis_name="c", subcore_axis_name="s")

    @functools.partial(
        pl.kernel,
        mesh=mesh,
        out_type=jax.ShapeDtypeStruct((E_PAD, H), jnp.float32),
        scratch_types=[
            pltpu.VMEM((2, jj, 128), jnp.int32),
            pltpu.VMEM((2, CH, H), jnp.float32),
            pltpu.VMEM((CH, H), jnp.float32),
            pltpu.SemaphoreType.DMA,
            pltpu.SemaphoreType.DMA,
        ],
        compiler_params=pltpu.CompilerParams(use_tc_tiling_on_sc=False),
    )
    def k(table_hbm, e_hbm, idx_hbm, out_hbm, idx_v, rows_v, e_v, semg, seme):
        c = lax.axis_index("c")
        s = lax.axis_index("s")
        wid = c * NS + s
        base = wid * epw
        irow = wid * (epw // 128)

        def load_idx(i, b):
            pltpu.sync_copy(idx_hbm.at[pl.ds(irow + i * jj, jj)], idx_v.at[b])

        def fire_h(b):
            for j in range(jj):
                pltpu.async_copy(table_hbm.at[idx_v.at[b, j]],
                                 rows_v.at[b, pl.ds(j * 128, 128)], semg)

        def drain_h(b):
            for j in range(jj):
                pltpu.make_async_copy(table_hbm.at[idx_v.at[b, j]],
                                      rows_v.at[b, pl.ds(j * 128, 128)],
                                      semg).wait()

        def fire_e(i):
            pltpu.async_copy(e_hbm.at[pl.ds(base + i * CH, CH)], e_v, seme)

        def drain_e(i):
            pltpu.make_async_copy(e_hbm.at[pl.ds(base + i * CH, CH)], e_v,
                                  seme).wait()

        def relu_add(b):
            def row2(r, carry):
                r0 = 2 * r
                for dr in range(2):
                    for q in range(0, H, 16):
                        hv = rows_v[b, r0 + dr, pl.ds(q, 16)]
                        ev = e_v[r0 + dr, pl.ds(q, 16)]
                        rows_v[b, r0 + dr, pl.ds(q, 16)] = (
                            jnp.maximum(hv + ev, 0.0))
                return carry

            lax.fori_loop(0, CH // 2, row2, 0)

        def store(i, b):
            pltpu.sync_copy(rows_v.at[b], out_hbm.at[pl.ds(base + i * CH, CH)])

        load_idx(0, 0)
        fire_h(0)
        fire_e(0)

        def body2(kk, carry):
            i0 = 2 * kk
            load_idx(i0 + 1, 1)
            fire_h(1)
            drain_h(0)
            drain_e(i0)
            relu_add(0)
            fire_e(i0 + 1)
            store(i0, 0)

            @pl.when(kk < nchh - 1)
            def _():
                load_idx(i0 + 2, 0)
                fire_h(0)

            drain_h(1)
            drain_e(i0 + 1)
            relu_add(1)

            @pl.when(kk < nchh - 1)
            def _():
                fire_e(i0 + 2)

            store(i0 + 1, 1)
            return carry

        lax.fori_loop(0, nchh, body2, 0)

    return k(table, e, idx2)


def _scatter_add(msg, dst2, zeros_tbl):
    ept = E_PAD // NS
    nch = ept // CHS
    nchh = nch // 2
    jj = CHS // 128
    rpt = TBL // NS
    mesh = plsc.VectorSubcoreMesh(core_axis_name="c", subcore_axis_name="s")

    @functools.partial(
        pl.kernel,
        mesh=mesh,
        out_type=jax.ShapeDtypeStruct((NC, TBL, HH), jnp.float32),
        scratch_types=[
            pltpu.VMEM((2, jj, 128), jnp.int32),
            pltpu.VMEM((2, CHS, HH), jnp.float32),
            pltpu.VMEM_SHARED((TBL, HH), jnp.float32),
            pltpu.SemaphoreType.DMA,
        ],
        compiler_params=pltpu.CompilerParams(use_tc_tiling_on_sc=False),
    )
    def k(msg_hbm, dst_hbm, z_hbm, out_hbm, idx_v, rows_v, table_sh, seml):
        c = lax.axis_index("c")
        s = lax.axis_index("s")
        pltpu.sync_copy(z_hbm.at[pl.ds(s * rpt, rpt)],
                        table_sh.at[pl.ds(s * rpt, rpt)])
        plsc.subcore_barrier()
        col = c * HH

        def fire_load(i, b):
            pltpu.async_copy(
                msg_hbm.at[pl.ds(s * ept + i * CHS, CHS), pl.ds(col, HH)],
                rows_v.at[b], seml)
            pltpu.async_copy(dst_hbm.at[pl.ds(s * (ept // 128) + i * jj, jj)],
                             idx_v.at[b], seml)

        def wait_load(i, b):
            pltpu.make_async_copy(
                msg_hbm.at[pl.ds(s * ept + i * CHS, CHS), pl.ds(col, HH)],
                rows_v.at[b], seml).wait()
            pltpu.make_async_copy(
                dst_hbm.at[pl.ds(s * (ept // 128) + i * jj, jj)],
                idx_v.at[b], seml).wait()

        def process(b):
            for j in range(jj):
                pltpu.sync_copy(rows_v.at[b, pl.ds(j * 128, 128)],
                                table_sh.at[idx_v.at[b, j]], add=True)

        fire_load(0, 0)

        def body2(kk, carry):
            i0 = 2 * kk
            fire_load(i0 + 1, 1)
            wait_load(i0, 0)
            process(0)

            @pl.when(kk < nchh - 1)
            def _():
                fire_load(i0 + 2, 0)

            wait_load(i0 + 1, 1)
            process(1)
            return carry

        lax.fori_loop(0, nchh, body2, 0)
        plsc.subcore_barrier()
        pltpu.sync_copy(table_sh.at[pl.ds(s * rpt, rpt)],
                        out_hbm.at[c, pl.ds(s * rpt, rpt)])

    return k(msg, dst2, zeros_tbl)


def _full(shape):
    return pl.BlockSpec(shape, lambda i: tuple(0 for _ in shape))


def _mlp_ln(xin, w1t, b1, w2t, b2, g, be, br, relu_out=False):
    n, d = xin.shape

    def body(x_ref, w1_ref, b1_ref, w2_ref, b2_ref, g_ref, be_ref, o_ref):
        h = jnp.maximum(x_ref[...] @ w1_ref[...] + b1_ref[...], 0.0)
        h = h @ w2_ref[...] + b2_ref[...]
        m = jnp.mean(h, axis=-1, keepdims=True)
        v = jnp.mean((h - m) ** 2, axis=-1, keepdims=True)
        o = (h - m) * lax.rsqrt(v + 1e-5) * g_ref[...] + be_ref[...]
        if relu_out:
            o = jnp.maximum(o, 0.0)
        o_ref[...] = o

    return pl.pallas_call(
        body,
        grid=(n // br,),
        in_specs=[
            pl.BlockSpec((br, d), lambda i: (i, 0)),
            _full((d, H)), _full((1, H)), _full((H, H)),
            _full((1, H)), _full((1, H)), _full((1, H)),
        ],
        out_specs=pl.BlockSpec((br, H), lambda i: (i, 0)),
        out_shape=jax.ShapeDtypeStruct((n, H), jnp.float32),
    )(xin, w1t, b1, w2t, b2, g, be)


def _gine_update(h, aggr, eps1, w1t, b1, w2t, b2, g, be, relu_out):
    br = 2000

    def body(h_ref, a_ref, e1_ref, w1_ref, b1_ref, w2_ref, b2_ref,
             g_ref, be_ref, o_ref):
        z = h_ref[...] * e1_ref[...] + a_ref[...]
        z = jnp.maximum(z @ w1_ref[...] + b1_ref[...], 0.0)
        z = z @ w2_ref[...] + b2_ref[...]
        m = jnp.mean(z, axis=-1, keepdims=True)
        v = jnp.mean((z - m) ** 2, axis=-1, keepdims=True)
        o = (z - m) * lax.rsqrt(v + 1e-5) * g_ref[...] + be_ref[...]
        if relu_out:
            o = jnp.maximum(o, 0.0)
        o_ref[...] = o

    return pl.pallas_call(
        body,
        grid=(N // br,),
        in_specs=[
            pl.BlockSpec((br, H), lambda i: (i, 0)),
            pl.BlockSpec((br, H), lambda i: (i, 0)),
            _full((1, H)), _full((H, H)), _full((1, H)),
            _full((H, H)), _full((1, H)), _full((1, H)), _full((1, H)),
        ],
        out_specs=pl.BlockSpec((br, H), lambda i: (i, 0)),
        out_shape=jax.ShapeDtypeStruct((N, H), jnp.float32),
    )(h, aggr, eps1, w1t, b1, w2t, b2, g, be)


def _pool_gp(h, gpwt, gpb, gpg, gpbe, wc, epb1):
    br = 2000
    steps = N // br

    def body(h_ref, gpw_ref, gpb_ref, gpg_ref, gpbe_ref, wc_ref, b1_ref,
             c0_ref, acc_ref):
        i = pl.program_id(0)

        @pl.when(i == 0)
        def _():
            acc_ref[...] = jnp.zeros_like(acc_ref)

        acc_ref[...] += jnp.sum(h_ref[...], axis=0, keepdims=True)

        @pl.when(i == steps - 1)
        def _():
            gm = acc_ref[...] * (1.0 / N)
            t = jnp.maximum(gm @ gpw_ref[...] + gpb_ref[...], 0.0)
            m = jnp.mean(t, axis=-1, keepdims=True)
            v = jnp.mean((t - m) ** 2, axis=-1, keepdims=True)
            gg = (t - m) * lax.rsqrt(v + 1e-5) * gpg_ref[...] + gpbe_ref[...]
            c0_ref[...] = gg @ wc_ref[...] + b1_ref[...]

    return pl.pallas_call(
        body,
        grid=(steps,),
        in_specs=[
            pl.BlockSpec((br, H), lambda i: (i, 0)),
            _full((H, H)), _full((1, H)), _full((1, H)), _full((1, H)),
            _full((H, 2 * H)), _full((1, 2 * H)),
        ],
        out_specs=_full((1, 2 * H)),
        out_shape=jax.ShapeDtypeStruct((1, 2 * H), jnp.float32),
        scratch_shapes=[pltpu.VMEM((1, H), jnp.float32)],
    )(h, gpwt, gpb, gpg, gpbe, wc, epb1)


def _predictor(s2, d2, e, c0, wa, wb, wd, w2t, b2, w3r, b3):
    br = 2048

    def body(s_ref, d_ref, e_ref, c0_ref, wa_ref, wb_ref, wd_ref,
             w2_ref, b2_ref, w3_ref, b3_ref, o_ref):
        sf = s_ref[...].astype(jnp.float32)
        df = d_ref[...].astype(jnp.float32)
        z1 = (sf @ wa_ref[...] + df @ wb_ref[...]
              + e_ref[...] @ wd_ref[...] + c0_ref[...])
        z1 = jnp.tanh(z1)
        z2 = jnp.tanh(z1 @ w2_ref[...] + b2_ref[...])
        sc = jnp.sum(z2 * w3_ref[...], axis=-1, keepdims=True) + b3_ref[...]
        o_ref[...] = jax.nn.sigmoid(sc)

    return pl.pallas_call(
        body,
        grid=(E_PAD // br,),
        in_specs=[
            pl.BlockSpec((br, H), lambda i: (i, 0)),
            pl.BlockSpec((br, H), lambda i: (i, 0)),
            pl.BlockSpec((br, H), lambda i: (i, 0)),
            _full((1, 2 * H)), _full((H, 2 * H)), _full((H, 2 * H)),
            _full((H, 2 * H)), _full((2 * H, H)), _full((1, H)),
            _full((1, H)), _full((1, 1)),
        ],
        out_specs=pl.BlockSpec((br, 1), lambda i: (i, 0)),
        out_shape=jax.ShapeDtypeStruct((E_PAD, 1), jnp.float32),
    )(s2, d2, e, c0, wa, wb, wd, w2t, b2, w3r, b3)


def kernel(x, edge_index, edge_attr, batch, params):
    p = params
    r1 = lambda a: a.reshape(1, -1)
    pad = E_PAD - E
    src = edge_index[0]
    dst = edge_index[1]
    src2 = jnp.concatenate([src, jnp.zeros((pad,), jnp.int32)]).reshape(E_PAD // 128, 128)
    dstg2 = jnp.concatenate([dst, jnp.zeros((pad,), jnp.int32)]).reshape(E_PAD // 128, 128)
    dsts2 = jnp.concatenate([dst, jnp.full((pad,), N, jnp.int32)]).reshape(E_PAD // 128, 128)
    ea_pad = jnp.pad(edge_attr, ((0, pad), (0, 0)))
    zeros_tbl = jnp.zeros((TBL, HH), jnp.float32)

    h = _mlp_ln(x, p['ne_W1'].T, r1(p['ne_b1']), p['ne_W2'].T, r1(p['ne_b2']),
                r1(p['ne_g']), r1(p['ne_be']), br=2000)
    e = _mlp_ln(ea_pad, p['ee_W1'].T, r1(p['ee_b1']), p['ee_W2'].T, r1(p['ee_b2']),
                r1(p['ee_g']), r1(p['ee_be']), br=2048)

    for li, l in enumerate(('l0', 'l1')):
        msg = _gather_msg(h, e, src2)
        agg = _scatter_add(msg, dsts2, zeros_tbl)
        aggr = jnp.concatenate([agg[0, :N], agg[1, :N]], axis=1)
        eps1 = r1(jnp.broadcast_to(1.0 + p[l + '_eps'], (H,)))
        h = _gine_update(h, aggr, eps1, p[l + '_W1'].T, r1(p[l + '_b1']),
                         p[l + '_W2'].T, r1(p[l + '_b2']),
                         r1(p[l + '_g']), r1(p[l + '_be']), relu_out=(li == 0))

    w1t = p['ep_W1'].T
    c0 = _pool_gp(h, p['gp_W'].T, r1(p['gp_b']), r1(p['gp_g']), r1(p['gp_be']),
                  w1t[2 * H:3 * H], r1(p['ep_b1']))
    sd = _gather_rows(h.astype(jnp.bfloat16),
                      jnp.concatenate([src2, dstg2]), 2 * E_PAD)
    s2 = sd[:E_PAD]
    d2 = sd[E_PAD:]
    out = _predictor(s2, d2, e, c0, w1t[:H], w1t[H:2 * H], w1t[3 * H:],
                     p['ep_W2'].T, r1(p['ep_b2']), r1(p['ep_W3']), r1(p['ep_b3']))
    return out[:E]


# speedup vs baseline: 1.4168x; 1.0304x over previous
import functools

import jax
import jax.numpy as jnp
from jax import lax
from jax.experimental import pallas as pl
from jax.experimental.pallas import tpu as pltpu
from jax.experimental.pallas import tpu_sc as plsc

N = 50000
E = 800000
H = 64
E_PAD = 819200
NC, NS = 2, 16
NW = NC * NS
HH = H // 2
TBL = 50016
CH = 512
CHS = 256


def _gather_rows(table, idx2, n_rows):
    epw = n_rows // NW
    nch = epw // CH
    nchh = nch // 2
    jj = CH // 128
    dt = table.dtype
    mesh = plsc.VectorSubcoreMesh(core_axis_name="c", subcore_axis_name="s")

    @functools.partial(
        pl.kernel,
        mesh=mesh,
        out_type=jax.ShapeDtypeStruct((n_rows, H), dt),
        scratch_types=[
            pltpu.VMEM((2, jj, 128), jnp.int32),
            pltpu.VMEM((2, CH, H), dt),
            pltpu.SemaphoreType.DMA,
        ],
        compiler_params=pltpu.CompilerParams(use_tc_tiling_on_sc=False),
    )
    def k(table_hbm, idx_hbm, out_hbm, idx_v, rows_v, semg):
        c = lax.axis_index("c")
        s = lax.axis_index("s")
        wid = s * NC + c
        base = wid * epw
        irow = wid * (epw // 128)

        def load_idx(i, b):
            pltpu.sync_copy(idx_hbm.at[pl.ds(irow + i * jj, jj)], idx_v.at[b])

        def fire(b):
            for j in range(jj):
                pltpu.async_copy(table_hbm.at[idx_v.at[b, j]],
                                 rows_v.at[b, pl.ds(j * 128, 128)], semg)

        def drain(b):
            for j in range(jj):
                pltpu.make_async_copy(table_hbm.at[idx_v.at[b, j]],
                                      rows_v.at[b, pl.ds(j * 128, 128)],
                                      semg).wait()

        def store(i, b):
            pltpu.sync_copy(rows_v.at[b], out_hbm.at[pl.ds(base + i * CH, CH)])

        load_idx(0, 0)
        fire(0)

        def body2(kk, carry):
            i0 = 2 * kk
            load_idx(i0 + 1, 1)
            drain(0)
            fire(1)
            store(i0, 0)

            @pl.when(kk < nchh - 1)
            def _():
                load_idx(i0 + 2, 0)
                fire(0)

            drain(1)
            store(i0 + 1, 1)
            return carry

        lax.fori_loop(0, nchh, body2, 0)

    return k(table, idx2)


def _gather_msg(table, e, idx2):
    epw = E_PAD // NW
    nch = epw // CH
    nchh = nch // 2
    jj = CH // 128
    mesh = plsc.VectorSubcoreMesh(core_axis_name="c", subcore_axis_name="s")

    @functools.partial(
        pl.kernel,
        mesh=mesh,
        out_type=jax.ShapeDtypeStruct((E_PAD, H), jnp.float32),
        scratch_types=[
            pltpu.VMEM((2, jj, 128), jnp.int32),
            pltpu.VMEM((2, CH, H), jnp.float32),
            pltpu.VMEM((CH, H), jnp.float32),
            pltpu.SemaphoreType.DMA,
            pltpu.SemaphoreType.DMA,
        ],
        compiler_params=pltpu.CompilerParams(use_tc_tiling_on_sc=False),
    )
    def k(table_hbm, e_hbm, idx_hbm, out_hbm, idx_v, rows_v, e_v, semg, seme):
        c = lax.axis_index("c")
        s = lax.axis_index("s")
        wid = c * NS + s
        base = wid * epw
        irow = wid * (epw // 128)

        def load_idx(i, b):
            pltpu.sync_copy(idx_hbm.at[pl.ds(irow + i * jj, jj)], idx_v.at[b])

        def fire_h(b):
            for j in range(jj):
                pltpu.async_copy(table_hbm.at[idx_v.at[b, j]],
                                 rows_v.at[b, pl.ds(j * 128, 128)], semg)

        def drain_h(b):
            for j in range(jj):
                pltpu.make_async_copy(table_hbm.at[idx_v.at[b, j]],
                                      rows_v.at[b, pl.ds(j * 128, 128)],
                                      semg).wait()

        def fire_e(i):
            pltpu.async_copy(e_hbm.at[pl.ds(base + i * CH, CH)], e_v, seme)

        def drain_e(i):
            pltpu.make_async_copy(e_hbm.at[pl.ds(base + i * CH, CH)], e_v,
                                  seme).wait()

        def relu_add(b):
            def row2(r, carry):
                r0 = 2 * r
                for dr in range(2):
                    for q in range(0, H, 16):
                        hv = rows_v[b, r0 + dr, pl.ds(q, 16)]
                        ev = e_v[r0 + dr, pl.ds(q, 16)]
                        rows_v[b, r0 + dr, pl.ds(q, 16)] = (
                            jnp.maximum(hv + ev, 0.0))
                return carry

            lax.fori_loop(0, CH // 2, row2, 0)

        def store(i, b):
            pltpu.sync_copy(rows_v.at[b], out_hbm.at[pl.ds(base + i * CH, CH)])

        load_idx(0, 0)
        fire_h(0)
        fire_e(0)

        def body2(kk, carry):
            i0 = 2 * kk
            load_idx(i0 + 1, 1)
            fire_h(1)
            drain_h(0)
            drain_e(i0)
            relu_add(0)
            fire_e(i0 + 1)
            store(i0, 0)

            @pl.when(kk < nchh - 1)
            def _():
                load_idx(i0 + 2, 0)
                fire_h(0)

            drain_h(1)
            drain_e(i0 + 1)
            relu_add(1)

            @pl.when(kk < nchh - 1)
            def _():
                fire_e(i0 + 2)

            store(i0 + 1, 1)
            return carry

        lax.fori_loop(0, nchh, body2, 0)

    return k(table, e, idx2)


def _scatter_add(msg, dst2, zeros_tbl):
    ept = E_PAD // NS
    nch = ept // CHS
    nchh = nch // 2
    jj = CHS // 128
    rpt = TBL // NS
    mesh = plsc.VectorSubcoreMesh(core_axis_name="c", subcore_axis_name="s")

    @functools.partial(
        pl.kernel,
        mesh=mesh,
        out_type=jax.ShapeDtypeStruct((NC, TBL, HH), jnp.float32),
        scratch_types=[
            pltpu.VMEM((2, jj, 128), jnp.int32),
            pltpu.VMEM((2, CHS, HH), jnp.float32),
            pltpu.VMEM_SHARED((TBL, HH), jnp.float32),
            pltpu.SemaphoreType.DMA,
        ],
        compiler_params=pltpu.CompilerParams(use_tc_tiling_on_sc=False),
    )
    def k(msg_hbm, dst_hbm, z_hbm, out_hbm, idx_v, rows_v, table_sh, seml):
        c = lax.axis_index("c")
        s = lax.axis_index("s")
        pltpu.sync_copy(z_hbm.at[pl.ds(s * rpt, rpt)],
                        table_sh.at[pl.ds(s * rpt, rpt)])
        plsc.subcore_barrier()
        col = c * HH

        def fire_load(i, b):
            pltpu.async_copy(
                msg_hbm.at[pl.ds(s * ept + i * CHS, CHS), pl.ds(col, HH)],
                rows_v.at[b], seml)
            pltpu.async_copy(dst_hbm.at[pl.ds(s * (ept // 128) + i * jj, jj)],
                             idx_v.at[b], seml)

        def wait_load(i, b):
            pltpu.make_async_copy(
                msg_hbm.at[pl.ds(s * ept + i * CHS, CHS), pl.ds(col, HH)],
                rows_v.at[b], seml).wait()
            pltpu.make_async_copy(
                dst_hbm.at[pl.ds(s * (ept // 128) + i * jj, jj)],
                idx_v.at[b], seml).wait()

        def process(b):
            for j in range(jj):
                pltpu.sync_copy(rows_v.at[b, pl.ds(j * 128, 128)],
                                table_sh.at[idx_v.at[b, j]], add=True)

        fire_load(0, 0)

        def body2(kk, carry):
            i0 = 2 * kk
            fire_load(i0 + 1, 1)
            wait_load(i0, 0)
            process(0)

            @pl.when(kk < nchh - 1)
            def _():
                fire_load(i0 + 2, 0)

            wait_load(i0 + 1, 1)
            process(1)
            return carry

        lax.fori_loop(0, nchh, body2, 0)
        plsc.subcore_barrier()
        pltpu.sync_copy(table_sh.at[pl.ds(s * rpt, rpt)],
                        out_hbm.at[c, pl.ds(s * rpt, rpt)])

    return k(msg, dst2, zeros_tbl)


def _full(shape):
    return pl.BlockSpec(shape, lambda i: tuple(0 for _ in shape))


def _mlp_ln(xin, w1t, b1, w2t, b2, g, be, br, relu_out=False):
    n, d = xin.shape

    def body(x_ref, w1_ref, b1_ref, w2_ref, b2_ref, g_ref, be_ref, o_ref):
        h = jnp.maximum(x_ref[...] @ w1_ref[...] + b1_ref[...], 0.0)
        h = h @ w2_ref[...] + b2_ref[...]
        m = jnp.mean(h, axis=-1, keepdims=True)
        v = jnp.mean((h - m) ** 2, axis=-1, keepdims=True)
        o = (h - m) * lax.rsqrt(v + 1e-5) * g_ref[...] + be_ref[...]
        if relu_out:
            o = jnp.maximum(o, 0.0)
        o_ref[...] = o

    return pl.pallas_call(
        body,
        grid=(n // br,),
        in_specs=[
            pl.BlockSpec((br, d), lambda i: (i, 0)),
            _full((d, H)), _full((1, H)), _full((H, H)),
            _full((1, H)), _full((1, H)), _full((1, H)),
        ],
        out_specs=pl.BlockSpec((br, H), lambda i: (i, 0)),
        out_shape=jax.ShapeDtypeStruct((n, H), jnp.float32),
    )(xin, w1t, b1, w2t, b2, g, be)


def _gine_update(h, aggr, eps1, w1t, b1, w2t, b2, g, be, relu_out):
    br = 2000

    def body(h_ref, a_ref, e1_ref, w1_ref, b1_ref, w2_ref, b2_ref,
             g_ref, be_ref, o_ref):
        z = h_ref[...] * e1_ref[...] + a_ref[...]
        z = jnp.maximum(z @ w1_ref[...] + b1_ref[...], 0.0)
        z = z @ w2_ref[...] + b2_ref[...]
        m = jnp.mean(z, axis=-1, keepdims=True)
        v = jnp.mean((z - m) ** 2, axis=-1, keepdims=True)
        o = (z - m) * lax.rsqrt(v + 1e-5) * g_ref[...] + be_ref[...]
        if relu_out:
            o = jnp.maximum(o, 0.0)
        o_ref[...] = o

    return pl.pallas_call(
        body,
        grid=(N // br,),
        in_specs=[
            pl.BlockSpec((br, H), lambda i: (i, 0)),
            pl.BlockSpec((br, H), lambda i: (i, 0)),
            _full((1, H)), _full((H, H)), _full((1, H)),
            _full((H, H)), _full((1, H)), _full((1, H)), _full((1, H)),
        ],
        out_specs=pl.BlockSpec((br, H), lambda i: (i, 0)),
        out_shape=jax.ShapeDtypeStruct((N, H), jnp.float32),
    )(h, aggr, eps1, w1t, b1, w2t, b2, g, be)


def _pool_gp(h, gpwt, gpb, gpg, gpbe, wc, epb1):
    br = 2000
    steps = N // br

    def body(h_ref, gpw_ref, gpb_ref, gpg_ref, gpbe_ref, wc_ref, b1_ref,
             c0_ref, acc_ref):
        i = pl.program_id(0)

        @pl.when(i == 0)
        def _():
            acc_ref[...] = jnp.zeros_like(acc_ref)

        acc_ref[...] += jnp.sum(h_ref[...], axis=0, keepdims=True)

        @pl.when(i == steps - 1)
        def _():
            gm = acc_ref[...] * (1.0 / N)
            t = jnp.maximum(gm @ gpw_ref[...] + gpb_ref[...], 0.0)
            m = jnp.mean(t, axis=-1, keepdims=True)
            v = jnp.mean((t - m) ** 2, axis=-1, keepdims=True)
            gg = (t - m) * lax.rsqrt(v + 1e-5) * gpg_ref[...] + gpbe_ref[...]
            c0_ref[...] = gg @ wc_ref[...] + b1_ref[...]

    return pl.pallas_call(
        body,
        grid=(steps,),
        in_specs=[
            pl.BlockSpec((br, H), lambda i: (i, 0)),
            _full((H, H)), _full((1, H)), _full((1, H)), _full((1, H)),
            _full((H, 2 * H)), _full((1, 2 * H)),
        ],
        out_specs=_full((1, 2 * H)),
        out_shape=jax.ShapeDtypeStruct((1, 2 * H), jnp.float32),
        scratch_shapes=[pltpu.VMEM((1, H), jnp.float32)],
    )(h, gpwt, gpb, gpg, gpbe, wc, epb1)


def _predictor(sd, e, c0, wa, wb, wd, w2t, b2, w3r, b3):
    br = 2048

    def body(s_ref, d_ref, e_ref, c0_ref, wa_ref, wb_ref, wd_ref,
             w2_ref, b2_ref, w3_ref, b3_ref, o_ref):
        sf = s_ref[...].astype(jnp.float32)
        df = d_ref[...].astype(jnp.float32)
        z1 = (sf @ wa_ref[...] + df @ wb_ref[...]
              + e_ref[...] @ wd_ref[...] + c0_ref[...])
        z1 = jnp.tanh(z1)
        z2 = jnp.tanh(z1 @ w2_ref[...] + b2_ref[...])
        sc = jnp.sum(z2 * w3_ref[...], axis=-1, keepdims=True) + b3_ref[...]
        o_ref[...] = jax.nn.sigmoid(sc)

    return pl.pallas_call(
        body,
        grid=(E_PAD // br,),
        in_specs=[
            pl.BlockSpec((br, H), lambda i: (i, 0)),
            pl.BlockSpec((br, H), lambda i: (E_PAD // br + i, 0)),
            pl.BlockSpec((br, H), lambda i: (i, 0)),
            _full((1, 2 * H)), _full((H, 2 * H)), _full((H, 2 * H)),
            _full((H, 2 * H)), _full((2 * H, H)), _full((1, H)),
            _full((1, H)), _full((1, 1)),
        ],
        out_specs=pl.BlockSpec((br, 1), lambda i: (i, 0)),
        out_shape=jax.ShapeDtypeStruct((E_PAD, 1), jnp.float32),
    )(sd, sd, e, c0, wa, wb, wd, w2t, b2, w3r, b3)


def kernel(x, edge_index, edge_attr, batch, params):
    p = params
    r1 = lambda a: a.reshape(1, -1)
    pad = E_PAD - E
    src = edge_index[0]
    dst = edge_index[1]
    src2 = jnp.concatenate([src, jnp.zeros((pad,), jnp.int32)]).reshape(E_PAD // 128, 128)
    dstg2 = jnp.concatenate([dst, jnp.zeros((pad,), jnp.int32)]).reshape(E_PAD // 128, 128)
    dsts2 = jnp.concatenate([dst, jnp.full((pad,), N, jnp.int32)]).reshape(E_PAD // 128, 128)
    ea_pad = jnp.pad(edge_attr, ((0, pad), (0, 0)))
    zeros_tbl = jnp.zeros((TBL, HH), jnp.float32)

    h = _mlp_ln(x, p['ne_W1'].T, r1(p['ne_b1']), p['ne_W2'].T, r1(p['ne_b2']),
                r1(p['ne_g']), r1(p['ne_be']), br=2000)
    e = _mlp_ln(ea_pad, p['ee_W1'].T, r1(p['ee_b1']), p['ee_W2'].T, r1(p['ee_b2']),
                r1(p['ee_g']), r1(p['ee_be']), br=2048)

    for li, l in enumerate(('l0', 'l1')):
        msg = _gather_msg(h, e, src2)
        agg = _scatter_add(msg, dsts2, zeros_tbl)
        aggr = jnp.concatenate([agg[0, :N], agg[1, :N]], axis=1)
        eps1 = r1(jnp.broadcast_to(1.0 + p[l + '_eps'], (H,)))
        h = _gine_update(h, aggr, eps1, p[l + '_W1'].T, r1(p[l + '_b1']),
                         p[l + '_W2'].T, r1(p[l + '_b2']),
                         r1(p[l + '_g']), r1(p[l + '_be']), relu_out=(li == 0))

    w1t = p['ep_W1'].T
    c0 = _pool_gp(h, p['gp_W'].T, r1(p['gp_b']), r1(p['gp_g']), r1(p['gp_be']),
                  w1t[2 * H:3 * H], r1(p['ep_b1']))
    sd = _gather_rows(h.astype(jnp.bfloat16),
                      jnp.concatenate([src2, dstg2]), 2 * E_PAD)
    out = _predictor(sd, e, c0, w1t[:H], w1t[H:2 * H], w1t[3 * H:],
                     p['ep_W2'].T, r1(p['ep_b2']), r1(p['ep_W3']), r1(p['ep_b3']))
    return out[:E]
